# baseline (speedup 1.0000x reference)
import functools

import jax
import jax.numpy as jnp
from jax import lax
from jax.experimental import pallas as pl
from jax.experimental.pallas import tpu as pltpu
from jax.experimental.pallas import tpu_sc as plsc

DIM = 64
LANES = 16
QUARTERS = DIM // LANES
NUM_CORES = 2
NUM_SUBCORES = 16
NW = NUM_CORES * NUM_SUBCORES
CHUNK = 128
WROWS = 160
PROWS = WROWS // 2
MARGIN = 1.0

_GATHER_DNUMS = lax.GatherDimensionNumbers(
    offset_dims=(), collapsed_slice_dims=(0,), start_index_map=(0,))


def _lane_shuffle(x, perm):
    return lax.gather(
        x, perm[:, None], _GATHER_DNUMS, slice_sizes=(1,),
        mode=lax.GatherScatterMode.PROMISE_IN_BOUNDS)


def _make_relayout(n_rows):
    assert n_rows % WROWS == 0
    nwin = n_rows // WROWS
    t_full = nwin // NW
    n_extra = nwin - t_full * NW
    assert t_full >= 8
    mesh = plsc.VectorSubcoreMesh(core_axis_name="c", subcore_axis_name="s")

    @functools.partial(
        pl.kernel,
        out_type=(
            jax.ShapeDtypeStruct((n_rows // 2, 2 * DIM), jnp.float32),
            jax.ShapeDtypeStruct((n_rows // 2, 2 * DIM), jnp.float32),
        ),
        mesh=mesh,
        scratch_types=[
            pltpu.VMEM((WROWS, DIM), jnp.float32),
            pltpu.VMEM((WROWS, DIM), jnp.float32),
            pltpu.VMEM((PROWS, 2 * DIM), jnp.float32),
            pltpu.VMEM((PROWS, 2 * DIM), jnp.float32),
            pltpu.SemaphoreType.DMA,
            pltpu.SemaphoreType.DMA,
        ],
    )
    def relayout_kernel(etab, rtab, epack, rpack, in0, in1, out0, out1,
                        rsem, wsem):
        wid = lax.axis_index("s") * NUM_CORES + lax.axis_index("c")
        ins = (in0, in1)
        outs = (out0, out1)

        def repack(src_v, dst_v):
            def body(p, carry):
                r = 2 * p
                for h in range(2):
                    for q in range(QUARTERS):
                        dst_v[p, pl.ds(h * DIM + q * LANES, LANES)] = (
                            src_v[r + h, pl.ds(q * LANES, LANES)])
                return carry
            lax.fori_loop(0, PROWS, body, 0)

        def phase(src, dst):
            def rofs(t):
                return pl.multiple_of((wid + t * NW) * WROWS, 8)

            def wofs(t):
                return pl.multiple_of((wid + t * NW) * PROWS, 8)

            def rd(t, b):
                return pltpu.async_copy(
                    src.at[pl.ds(rofs(t), WROWS)], ins[b % 2], rsem)

            def wr(t, b):
                return pltpu.async_copy(
                    outs[b % 2], dst.at[pl.ds(wofs(t), PROWS)], wsem)

            def wait_rd(b):
                pltpu.make_async_copy(
                    src.at[pl.ds(0, WROWS)], ins[b % 2], rsem).wait()

            def wait_wr(b):
                pltpu.make_async_copy(
                    outs[b % 2], dst.at[pl.ds(0, PROWS)], wsem).wait()

            def step(t, b, with_read, with_wwait):
                if with_wwait:
                    wait_wr(b)
                wait_rd(b)
                repack(ins[b % 2], outs[b % 2])
                wr(t, b)
                if with_read:
                    rd(t + 2, b)

            last = t_full - 1
            ring_end = 2 * ((last - 1) // 2)
            rd(0, 0)
            rd(1, 1)
            for t in range(4):
                step(t, t, with_read=True, with_wwait=t >= 2)

            def ring_body(t2, carry):
                for b in range(2):
                    step(t2 * 2 + b, b, with_read=True, with_wwait=True)
                return carry
            lax.fori_loop(2, ring_end // 2, ring_body, 0)

            for t in range(ring_end, last + 1):
                step(t, t, with_read=t + 2 <= last, with_wwait=True)
            for t in range(2):
                wait_wr(t)

            @pl.when(wid < n_extra)
            def _():
                w0 = pl.multiple_of((t_full * NW + wid) * WROWS, 8)
                p0 = pl.multiple_of((t_full * NW + wid) * PROWS, 8)
                pltpu.sync_copy(src.at[pl.ds(w0, WROWS)], in0)
                repack(in0, out0)
                pltpu.sync_copy(out0, dst.at[pl.ds(p0, PROWS)])

        phase(etab, epack)
        phase(rtab, rpack)

    return relayout_kernel


def _make_transe(B):
    assert B % NW == 0
    per_w = B // NW
    assert per_w % CHUNK == 0
    nch = per_w // CHUNK
    mesh = plsc.VectorSubcoreMesh(core_axis_name="c", subcore_axis_name="s")
    buf_shape = (CHUNK, 2 * DIM)

    @functools.partial(
        pl.kernel,
        out_type=jax.ShapeDtypeStruct((NW, LANES), jnp.float32),
        mesh=mesh,
        scratch_types=[
            pltpu.VMEM((6, nch, CHUNK), jnp.int32),
            pltpu.VMEM((6, nch, CHUNK), jnp.int32),
            pltpu.VMEM(buf_shape, jnp.float32),
            pltpu.VMEM(buf_shape, jnp.float32),
            pltpu.VMEM(buf_shape, jnp.float32),
            pltpu.VMEM(buf_shape, jnp.float32),
            pltpu.VMEM(buf_shape, jnp.float32),
            pltpu.VMEM(buf_shape, jnp.float32),
            pltpu.VMEM((LANES,), jnp.float32),
            pltpu.SemaphoreType.DMA,
        ],
    )
    def transe_kernel(pair_hbm, half_hbm, etab, rtab, out_hbm, pair_v,
                      half_v, bph, bpr, bpt, bnh, bnr, bnt, ovec, sem):
        wid = lax.axis_index("s") * NUM_CORES + lax.axis_index("c")
        pltpu.sync_copy(pair_hbm.at[wid], pair_v)
        pltpu.sync_copy(half_hbm.at[wid], half_v)
        bufs = (bph, bpr, bpt, bnh, bnr, bnt)
        tabs = (etab, rtab, etab, etab, rtab, etab)

        def chunk_body(c, loss_vec):
            copies = [
                pltpu.async_copy(tabs[j].at[pair_v.at[j, c]], bufs[j], sem)
                for j in range(6)
            ]
            for cp in copies:
                cp.wait()

            def group_body(g, lv):
                base = g * LANES
                half_vecs = [half_v[j, c, pl.ds(base, LANES)]
                             for j in range(6)]
                for k in range(LANES):
                    i = base + k
                    offs = [half_vecs[j][k] * DIM for j in range(6)]
                    gap = None
                    for q in range(QUARTERS):
                        qo = q * LANES
                        p = jnp.abs(
                            bph[i, pl.ds(offs[0] + qo, LANES)]
                            + bpr[i, pl.ds(offs[1] + qo, LANES)]
                            - bpt[i, pl.ds(offs[2] + qo, LANES)])
                        n = jnp.abs(
                            bnh[i, pl.ds(offs[3] + qo, LANES)]
                            + bnr[i, pl.ds(offs[4] + qo, LANES)]
                            - bnt[i, pl.ds(offs[5] + qo, LANES)])
                        gap = p - n if gap is None else gap + (p - n)
                    s = gap
                    for b in (8, 4, 2, 1):
                        perm = lax.iota(jnp.int32, LANES) ^ b
                        s = s + _lane_shuffle(s, perm)
                    hinge = jnp.maximum(MARGIN + s, 0.0)
                    lane_hit = lax.iota(jnp.int32, LANES) == k
                    lv = lv + jnp.where(lane_hit, hinge, 0.0)
                return lv

            return lax.fori_loop(0, CHUNK // LANES, group_body, loss_vec)

        loss_vec = lax.fori_loop(0, nch, chunk_body,
                                 jnp.zeros((LANES,), jnp.float32))
        ovec[...] = loss_vec
        pltpu.sync_copy(ovec, out_hbm.at[wid])

    return transe_kernel


def kernel(positive_triples, negative_triples, entity_embeddings,
           relation_embeddings):
    B = positive_triples.shape[0]
    n_rows = entity_embeddings.shape[0]
    per_w = B // NW
    nch = per_w // CHUNK
    idx = jnp.stack(
        [
            positive_triples[:, 0],
            positive_triples[:, 1],
            positive_triples[:, 2],
            negative_triples[:, 0],
            negative_triples[:, 1],
            negative_triples[:, 2],
        ],
        axis=0,
    )
    idx = idx.reshape(6, NW, nch, CHUNK).transpose(1, 0, 2, 3)
    pair = idx >> 1
    half = idx & 1
    epack, rpack = _make_relayout(n_rows)(entity_embeddings,
                                          relation_embeddings)
    partials = _make_transe(B)(pair, half, epack, rpack)
    return jnp.sum(partials) * (1.0 / B)


# speedup vs baseline: 1.0179x; 1.0179x over previous
import functools

import jax
import jax.numpy as jnp
from jax import lax
from jax.experimental import pallas as pl
from jax.experimental.pallas import tpu as pltpu
from jax.experimental.pallas import tpu_sc as plsc

DIM = 64
LANES = 16
QUARTERS = DIM // LANES
NUM_CORES = 2
NUM_SUBCORES = 16
NW = NUM_CORES * NUM_SUBCORES
CHUNK = 128
WROWS = 160
PROWS = WROWS // 2
MARGIN = 1.0

_GATHER_DNUMS = lax.GatherDimensionNumbers(
    offset_dims=(), collapsed_slice_dims=(0,), start_index_map=(0,))


def _lane_shuffle(x, perm):
    return lax.gather(
        x, perm[:, None], _GATHER_DNUMS, slice_sizes=(1,),
        mode=lax.GatherScatterMode.PROMISE_IN_BOUNDS)


def _make_relayout(n_rows):
    assert n_rows % WROWS == 0
    nwin = n_rows // WROWS
    t_full = nwin // NW
    n_extra = nwin - t_full * NW
    assert t_full >= 8
    mesh = plsc.VectorSubcoreMesh(core_axis_name="c", subcore_axis_name="s")

    @functools.partial(
        pl.kernel,
        out_type=(
            jax.ShapeDtypeStruct((n_rows // 2, 2 * DIM), jnp.float32),
            jax.ShapeDtypeStruct((n_rows // 2, 2 * DIM), jnp.float32),
        ),
        mesh=mesh,
        scratch_types=[
            pltpu.VMEM((WROWS, DIM), jnp.float32),
            pltpu.VMEM((WROWS, DIM), jnp.float32),
            pltpu.VMEM((PROWS, 2 * DIM), jnp.float32),
            pltpu.VMEM((PROWS, 2 * DIM), jnp.float32),
            pltpu.SemaphoreType.DMA,
            pltpu.SemaphoreType.DMA,
        ],
    )
    def relayout_kernel(etab, rtab, epack, rpack, in0, in1, out0, out1,
                        rsem, wsem):
        wid = lax.axis_index("s") * NUM_CORES + lax.axis_index("c")
        ins = (in0, in1)
        outs = (out0, out1)

        def repack(src_v, dst_v):
            def body(p4, carry):
                p0 = p4 * 4
                for dp in range(4):
                    p = p0 + dp
                    r = 2 * p
                    for h in range(2):
                        for q in range(QUARTERS):
                            dst_v[p, pl.ds(h * DIM + q * LANES, LANES)] = (
                                src_v[r + h, pl.ds(q * LANES, LANES)])
                return carry
            lax.fori_loop(0, PROWS // 4, body, 0)

        def phase(src, dst):
            def rofs(t):
                return pl.multiple_of((wid + t * NW) * WROWS, 8)

            def wofs(t):
                return pl.multiple_of((wid + t * NW) * PROWS, 8)

            def rd(t, b):
                return pltpu.async_copy(
                    src.at[pl.ds(rofs(t), WROWS)], ins[b % 2], rsem)

            def wr(t, b):
                return pltpu.async_copy(
                    outs[b % 2], dst.at[pl.ds(wofs(t), PROWS)], wsem)

            def wait_rd(b):
                pltpu.make_async_copy(
                    src.at[pl.ds(0, WROWS)], ins[b % 2], rsem).wait()

            def wait_wr(b):
                pltpu.make_async_copy(
                    outs[b % 2], dst.at[pl.ds(0, PROWS)], wsem).wait()

            def step(t, b, with_read, with_wwait):
                if with_wwait:
                    wait_wr(b)
                wait_rd(b)
                repack(ins[b % 2], outs[b % 2])
                wr(t, b)
                if with_read:
                    rd(t + 2, b)

            last = t_full - 1
            ring_end = 2 * ((last - 1) // 2)
            rd(0, 0)
            rd(1, 1)
            for t in range(4):
                step(t, t, with_read=True, with_wwait=t >= 2)

            def ring_body(t2, carry):
                for b in range(2):
                    step(t2 * 2 + b, b, with_read=True, with_wwait=True)
                return carry
            lax.fori_loop(2, ring_end // 2, ring_body, 0)

            for t in range(ring_end, last + 1):
                step(t, t, with_read=t + 2 <= last, with_wwait=True)
            for t in range(2):
                wait_wr(t)

            @pl.when(wid < n_extra)
            def _():
                w0 = pl.multiple_of((t_full * NW + wid) * WROWS, 8)
                p0 = pl.multiple_of((t_full * NW + wid) * PROWS, 8)
                pltpu.sync_copy(src.at[pl.ds(w0, WROWS)], in0)
                repack(in0, out0)
                pltpu.sync_copy(out0, dst.at[pl.ds(p0, PROWS)])

        phase(etab, epack)
        phase(rtab, rpack)

    return relayout_kernel


def _make_transe(B):
    assert B % NW == 0
    per_w = B // NW
    assert per_w % CHUNK == 0
    nch = per_w // CHUNK
    mesh = plsc.VectorSubcoreMesh(core_axis_name="c", subcore_axis_name="s")
    buf_shape = (CHUNK, 2 * DIM)

    @functools.partial(
        pl.kernel,
        out_type=jax.ShapeDtypeStruct((NW, LANES), jnp.float32),
        mesh=mesh,
        scratch_types=[
            pltpu.VMEM((6, nch, CHUNK), jnp.int32),
            pltpu.VMEM((6, nch, CHUNK), jnp.int32),
            pltpu.VMEM(buf_shape, jnp.float32),
            pltpu.VMEM(buf_shape, jnp.float32),
            pltpu.VMEM(buf_shape, jnp.float32),
            pltpu.VMEM(buf_shape, jnp.float32),
            pltpu.VMEM(buf_shape, jnp.float32),
            pltpu.VMEM(buf_shape, jnp.float32),
            pltpu.VMEM((LANES,), jnp.float32),
            pltpu.SemaphoreType.DMA,
        ],
    )
    def transe_kernel(pair_hbm, half_hbm, etab, rtab, out_hbm, pair_v,
                      half_v, bph, bpr, bpt, bnh, bnr, bnt, ovec, sem):
        wid = lax.axis_index("s") * NUM_CORES + lax.axis_index("c")
        pltpu.sync_copy(pair_hbm.at[wid], pair_v)
        pltpu.sync_copy(half_hbm.at[wid], half_v)
        bufs = (bph, bpr, bpt, bnh, bnr, bnt)
        tabs = (etab, rtab, etab, etab, rtab, etab)

        def chunk_body(c, loss_vec):
            copies = [
                pltpu.async_copy(tabs[j].at[pair_v.at[j, c]], bufs[j], sem)
                for j in range(6)
            ]
            for cp in copies:
                cp.wait()

            def group_body(g, lv):
                base = g * LANES
                half_vecs = [half_v[j, c, pl.ds(base, LANES)]
                             for j in range(6)]
                for k in range(LANES):
                    i = base + k
                    offs = [half_vecs[j][k] * DIM for j in range(6)]
                    gap = None
                    for q in range(QUARTERS):
                        qo = q * LANES
                        p = jnp.abs(
                            bph[i, pl.ds(offs[0] + qo, LANES)]
                            + bpr[i, pl.ds(offs[1] + qo, LANES)]
                            - bpt[i, pl.ds(offs[2] + qo, LANES)])
                        n = jnp.abs(
                            bnh[i, pl.ds(offs[3] + qo, LANES)]
                            + bnr[i, pl.ds(offs[4] + qo, LANES)]
                            - bnt[i, pl.ds(offs[5] + qo, LANES)])
                        gap = p - n if gap is None else gap + (p - n)
                    s = gap
                    for b in (8, 4, 2, 1):
                        perm = lax.iota(jnp.int32, LANES) ^ b
                        s = s + _lane_shuffle(s, perm)
                    hinge = jnp.maximum(MARGIN + s, 0.0)
                    lane_hit = lax.iota(jnp.int32, LANES) == k
                    lv = lv + jnp.where(lane_hit, hinge, 0.0)
                return lv

            return lax.fori_loop(0, CHUNK // LANES, group_body, loss_vec)

        loss_vec = lax.fori_loop(0, nch, chunk_body,
                                 jnp.zeros((LANES,), jnp.float32))
        ovec[...] = loss_vec
        pltpu.sync_copy(ovec, out_hbm.at[wid])

    return transe_kernel


def kernel(positive_triples, negative_triples, entity_embeddings,
           relation_embeddings):
    B = positive_triples.shape[0]
    n_rows = entity_embeddings.shape[0]
    per_w = B // NW
    nch = per_w // CHUNK
    idx = jnp.stack(
        [
            positive_triples[:, 0],
            positive_triples[:, 1],
            positive_triples[:, 2],
            negative_triples[:, 0],
            negative_triples[:, 1],
            negative_triples[:, 2],
        ],
        axis=0,
    )
    idx = idx.reshape(6, NW, nch, CHUNK).transpose(1, 0, 2, 3)
    pair = idx >> 1
    half = idx & 1
    epack, rpack = _make_relayout(n_rows)(entity_embeddings,
                                          relation_embeddings)
    partials = _make_transe(B)(pair, half, epack, rpack)
    return jnp.sum(partials) * (1.0 / B)


# speedup vs baseline: 2.9938x; 2.9410x over previous
import functools

import jax
import jax.numpy as jnp
from jax import lax
from jax.experimental import pallas as pl
from jax.experimental.pallas import tpu as pltpu
from jax.experimental.pallas import tpu_sc as plsc

DIM = 64
LANES = 16
QUARTERS = DIM // LANES
NUM_CORES = 2
NUM_SUBCORES = 16
NW = NUM_CORES * NUM_SUBCORES
GROUP = 16
SUBLANES = 8
MARGIN = 1.0

_GATHER_DNUMS = lax.GatherDimensionNumbers(
    offset_dims=(), collapsed_slice_dims=(0,), start_index_map=(0,))


def _lane_shuffle(x, perm):
    return lax.gather(
        x, perm[:, None], _GATHER_DNUMS, slice_sizes=(1,),
        mode=lax.GatherScatterMode.PROMISE_IN_BOUNDS)


def _make_transe(B):
    assert B % NW == 0
    per_w = B // NW
    assert per_w % GROUP == 0
    ngr = per_w // GROUP
    assert ngr % 2 == 0 and ngr >= 4
    mesh = plsc.VectorSubcoreMesh(core_axis_name="c", subcore_axis_name="s")

    @functools.partial(
        pl.kernel,
        out_type=jax.ShapeDtypeStruct((NW, LANES), jnp.float32),
        mesh=mesh,
        scratch_types=[
            pltpu.VMEM((6, ngr, GROUP), jnp.int32),
            pltpu.VMEM((6, ngr, GROUP), jnp.int32),
            pltpu.VMEM((2, 6, GROUP, DIM), jnp.float32),
            pltpu.VMEM((LANES,), jnp.float32),
            pltpu.SemaphoreType.DMA,
            pltpu.SemaphoreType.DMA,
        ],
    )
    def transe_kernel(blk_hbm, sub_hbm, etab, rtab, out_hbm, blk_v, sub_v,
                      rows, ovec, sem0, sem1):
        wid = lax.axis_index("s") * NUM_CORES + lax.axis_index("c")
        pltpu.sync_copy(blk_hbm.at[wid], blk_v)
        pltpu.sync_copy(sub_hbm.at[wid], sub_v)
        tabs = (etab, rtab, etab, etab, rtab, etab)
        sems = (sem0, sem1)

        def issue(g, b):
            blk_vecs = [blk_v[j, g, :] for j in range(6)]
            sub_vecs = [sub_v[j, g, :] for j in range(6)]
            for j in range(6):
                for k in range(GROUP):
                    pltpu.async_copy(
                        tabs[j].at[blk_vecs[j][k], sub_vecs[j][k]],
                        rows.at[b, j, k], sems[b])

        def drain(b):
            for j in range(6):
                for k in range(GROUP):
                    pltpu.make_async_copy(
                        tabs[j].at[0, 0], rows.at[b, j, k], sems[b]).wait()

        def compute(loss_vec, b):
            for k in range(GROUP):
                gap = None
                for q in range(QUARTERS):
                    sl = pl.ds(q * LANES, LANES)
                    p = jnp.abs(rows[b, 0, k, sl] + rows[b, 1, k, sl]
                                - rows[b, 2, k, sl])
                    n = jnp.abs(rows[b, 3, k, sl] + rows[b, 4, k, sl]
                                - rows[b, 5, k, sl])
                    gap = p - n if gap is None else gap + (p - n)
                s = gap
                for bb in (8, 4, 2, 1):
                    perm = lax.iota(jnp.int32, LANES) ^ bb
                    s = s + _lane_shuffle(s, perm)
                hinge = jnp.maximum(MARGIN + s, 0.0)
                lane_hit = lax.iota(jnp.int32, LANES) == k
                loss_vec = loss_vec + jnp.where(lane_hit, hinge, 0.0)
            return loss_vec

        issue(0, 0)

        def pair_body(g2, loss_vec):
            g = g2 * 2
            issue(g + 1, 1)
            drain(0)
            loss_vec = compute(loss_vec, 0)
            issue(g + 2, 0)
            drain(1)
            return compute(loss_vec, 1)

        loss_vec = lax.fori_loop(0, ngr // 2 - 1, pair_body,
                                 jnp.zeros((LANES,), jnp.float32))
        issue(ngr - 1, 1)
        drain(0)
        loss_vec = compute(loss_vec, 0)
        drain(1)
        loss_vec = compute(loss_vec, 1)

        ovec[...] = loss_vec
        pltpu.sync_copy(ovec, out_hbm.at[wid])

    return transe_kernel


def kernel(positive_triples, negative_triples, entity_embeddings,
           relation_embeddings):
    B = positive_triples.shape[0]
    per_w = B // NW
    ngr = per_w // GROUP
    idx = jnp.stack(
        [
            positive_triples[:, 0],
            positive_triples[:, 1],
            positive_triples[:, 2],
            negative_triples[:, 0],
            negative_triples[:, 1],
            negative_triples[:, 2],
        ],
        axis=0,
    )
    idx = idx.reshape(6, NW, ngr, GROUP).transpose(1, 0, 2, 3)
    blk = idx >> 3
    sub = idx & 7
    etab3 = entity_embeddings.reshape(-1, SUBLANES, DIM)
    rtab3 = relation_embeddings.reshape(-1, SUBLANES, DIM)
    partials = _make_transe(B)(blk, sub, etab3, rtab3)
    return jnp.sum(partials) * (1.0 / B)
